# Initial kernel scaffold; baseline (speedup 1.0000x reference)
import jax
import jax.numpy as jnp
from jax.experimental import pallas as pl


def kernel(x, adj_matrix, params):
    raise NotImplementedError("write your pallas kernel here")


# speedup vs baseline: 14.0600x; 14.0600x over previous
import functools

import jax
import jax.numpy as jnp
from jax import lax
from jax.experimental import pallas as pl
from jax.experimental.pallas import tpu as pltpu
from jax.experimental.pallas import tpu_sc as plsc

ALPHA = 0.05
NS = 16
NCORE = 2
CHUNK = 1000
ROWB = 2000


def _sc_degree(dst, zeros_n, ones_c):
    n = zeros_n.shape[0]
    e = dst.shape[0]
    rps = (n // (8 * NS)) * 8
    tail = n - rps * NS
    share = e // (NCORE * NS)
    nch = share // CHUNK
    mesh = plsc.VectorSubcoreMesh(core_axis_name="c", subcore_axis_name="s")

    @functools.partial(
        pl.kernel,
        out_type=jax.ShapeDtypeStruct((NCORE * n,), jnp.float32),
        mesh=mesh,
        scratch_types=[
            pltpu.VMEM_SHARED((n,), jnp.float32),
            pltpu.VMEM((CHUNK,), jnp.int32),
            pltpu.VMEM((CHUNK,), jnp.float32),
            pltpu.VMEM((rps,), jnp.float32),
        ],
        compiler_params=pltpu.CompilerParams(use_tc_tiling_on_sc=False),
    )
    def k(dst_h, zeros_h, ones_h, out_h, deg_sh, dst_v, ones_v, bounce):
        c = lax.axis_index("c")
        s = lax.axis_index("s")
        r0 = s * rps
        pltpu.sync_copy(zeros_h.at[pl.ds(0, rps)], bounce)
        pltpu.sync_copy(bounce, deg_sh.at[pl.ds(r0, rps)])
        if tail:
            @pl.when(s == NS - 1)
            def _():
                pltpu.sync_copy(bounce.at[pl.ds(0, tail)],
                                deg_sh.at[pl.ds(rps * NS, tail)])
        pltpu.sync_copy(ones_h, ones_v)
        plsc.subcore_barrier()
        base0 = (c * NS + s) * share

        def step(i, carry):
            base = base0 + i * CHUNK
            pltpu.sync_copy(dst_h.at[pl.ds(base, CHUNK)], dst_v)
            pltpu.sync_copy(ones_v, deg_sh.at[dst_v], add=True)
            return carry

        lax.fori_loop(0, nch, step, 0)
        plsc.subcore_barrier()
        pltpu.sync_copy(deg_sh.at[pl.ds(r0, rps)], bounce)
        pltpu.sync_copy(bounce, out_h.at[pl.ds(c * n + r0, rps)])
        if tail:
            @pl.when(s == NS - 1)
            def _():
                pltpu.sync_copy(deg_sh.at[pl.ds(rps * NS, tail)],
                                bounce.at[pl.ds(0, tail)])
                pltpu.sync_copy(bounce.at[pl.ds(0, tail)],
                                out_h.at[pl.ds(c * n + rps * NS, tail)])

    return k(dst, zeros_n, ones_c)


def _sc_prop(table, srcs2, dst, zeros16, n, feature_split):
    e = dst.shape[0]
    rps = (n // (8 * NS)) * 8
    tail = n - rps * NS
    if feature_split:
        share = e // NS
    else:
        share = e // (NCORE * NS)
    nch = share // CHUNK
    mesh = plsc.VectorSubcoreMesh(core_axis_name="c", subcore_axis_name="s")

    zb = 88
    nzb = rps // zb

    @functools.partial(
        pl.kernel,
        out_type=jax.ShapeDtypeStruct((NCORE, n, 16), jnp.float32),
        mesh=mesh,
        scratch_types=[
            pltpu.VMEM_SHARED((n, 16), jnp.float32),
            pltpu.VMEM((CHUNK,), jnp.int32),
            pltpu.VMEM((CHUNK,), jnp.int32),
            pltpu.VMEM((CHUNK, 16), jnp.float32),
            pltpu.VMEM((zb, 16), jnp.float32),
            pltpu.SemaphoreType.DMA,
        ],
        compiler_params=pltpu.CompilerParams(use_tc_tiling_on_sc=False),
    )
    def k(table_h, srcs_h, dst_h, zeros_h, out_h,
          acc_sh, src_v, dst_v, rows_v, bounce, sem):
        c = lax.axis_index("c")
        s = lax.axis_index("s")
        r0 = s * rps
        pltpu.sync_copy(zeros_h.at[pl.ds(0, zb)], bounce)

        def zstep(j, carry):
            pltpu.sync_copy(bounce, acc_sh.at[pl.ds(r0 + j * zb, zb)])
            return carry

        lax.fori_loop(0, nzb, zstep, 0)
        if tail:
            @pl.when(s == NS - 1)
            def _():
                pltpu.sync_copy(bounce.at[pl.ds(0, tail)],
                                acc_sh.at[pl.ds(rps * NS, tail)])
        plsc.subcore_barrier()
        if feature_split:
            dbase0 = s * share
            sbase0 = c * e + dbase0
        else:
            dbase0 = (c * NS + s) * share
            sbase0 = dbase0

        def step(i, carry):
            pltpu.sync_copy(srcs_h.at[pl.ds(sbase0 + i * CHUNK, CHUNK)], src_v)
            pltpu.sync_copy(dst_h.at[pl.ds(dbase0 + i * CHUNK, CHUNK)], dst_v)
            pltpu.async_copy(table_h.at[src_v], rows_v, sem).wait()
            pltpu.sync_copy(rows_v, acc_sh.at[dst_v], add=True)
            return carry

        lax.fori_loop(0, nch, step, 0)
        plsc.subcore_barrier()

        def ostep(j, carry):
            pltpu.sync_copy(acc_sh.at[pl.ds(r0 + j * zb, zb)], bounce)
            pltpu.sync_copy(bounce, out_h.at[c, pl.ds(r0 + j * zb, zb)])
            return carry

        lax.fori_loop(0, nzb, ostep, 0)
        if tail:
            @pl.when(s == NS - 1)
            def _():
                pltpu.sync_copy(acc_sh.at[pl.ds(rps * NS, tail)],
                                bounce.at[pl.ds(0, tail)])
                pltpu.sync_copy(bounce.at[pl.ds(0, tail)],
                                out_h.at[c, pl.ds(rps * NS, tail)])

    return k(table, srcs2, dst, zeros16)


def _tc_dinv(degp, n):
    m = n // 8
    d2 = degp.reshape(NCORE, m, 8)

    def body(d_ref, dinv_ref, dinv2_ref):
        deg = d_ref[0] + d_ref[1] + 1.0
        dinv = lax.rsqrt(deg)
        dinv_ref[...] = dinv
        dinv2_ref[...] = dinv * dinv

    out = pl.pallas_call(
        body,
        out_shape=[jax.ShapeDtypeStruct((m, 8), jnp.float32),
                   jax.ShapeDtypeStruct((m, 8), jnp.float32)],
    )(d2)
    return out[0].reshape(n, 1), out[1].reshape(n, 1)


def _tc_linstats(xin, w, b8):
    n, fi = xin.shape
    fo = w.shape[1]
    nb = n // ROWB

    def body(x_ref, w_ref, b_ref, y_ref, s1_ref, s2_ref):
        if fi > 1:
            xb = x_ref[...].astype(jnp.bfloat16).astype(jnp.float32)
            wb = w_ref[...].astype(jnp.bfloat16).astype(jnp.float32)
        else:
            xb = x_ref[...]
            wb = w_ref[...]
        yb = jnp.broadcast_to(b_ref[0:1, :], (ROWB, fo))
        for kk in range(fi):
            yb = yb + xb[:, kk:kk + 1] * wb[kk:kk + 1, :]
        y_ref[...] = yb

        @pl.when(pl.program_id(0) == 0)
        def _():
            s1_ref[...] = jnp.zeros_like(s1_ref)
            s2_ref[...] = jnp.zeros_like(s2_ref)

        s1_ref[...] += jnp.broadcast_to(
            jnp.sum(yb, axis=0, keepdims=True), (8, fo))
        s2_ref[...] += jnp.broadcast_to(
            jnp.sum(yb * yb, axis=0, keepdims=True), (8, fo))

    return pl.pallas_call(
        body,
        grid=(nb,),
        in_specs=[pl.BlockSpec((ROWB, fi), lambda i: (i, 0)),
                  pl.BlockSpec((fi, fo), lambda i: (0, 0)),
                  pl.BlockSpec((8, fo), lambda i: (0, 0))],
        out_specs=[pl.BlockSpec((ROWB, fo), lambda i: (i, 0)),
                   pl.BlockSpec((8, fo), lambda i: (0, 0)),
                   pl.BlockSpec((8, fo), lambda i: (0, 0))],
        out_shape=[jax.ShapeDtypeStruct((n, fo), jnp.float32),
                   jax.ShapeDtypeStruct((8, fo), jnp.float32),
                   jax.ShapeDtypeStruct((8, fo), jnp.float32)],
    )(xin, w, b8)


def _tc_bnact(y, scale8, shift8, a8):
    n, f = y.shape
    nb = n // ROWB

    def body(y_ref, s_ref, t_ref, a_ref, z_ref):
        z = y_ref[...] * s_ref[0:1, :] + t_ref[0:1, :]
        z_ref[...] = jnp.where(z >= 0, z, a_ref[0:1, :] * z)

    return pl.pallas_call(
        body,
        grid=(nb,),
        in_specs=[pl.BlockSpec((ROWB, f), lambda i: (i, 0)),
                  pl.BlockSpec((8, f), lambda i: (0, 0)),
                  pl.BlockSpec((8, f), lambda i: (0, 0)),
                  pl.BlockSpec((8, f), lambda i: (0, 0))],
        out_specs=pl.BlockSpec((ROWB, f), lambda i: (i, 0)),
        out_shape=jax.ShapeDtypeStruct((n, f), jnp.float32),
    )(y, scale8, shift8, a8)


def _tc_table(xin, dinv):
    n, f = xin.shape
    nb = n // ROWB

    def body(x_ref, d_ref, t_ref):
        t_ref[...] = d_ref[...] * x_ref[...]

    return pl.pallas_call(
        body,
        grid=(nb,),
        in_specs=[pl.BlockSpec((ROWB, f), lambda i: (i, 0)),
                  pl.BlockSpec((ROWB, 1), lambda i: (i, 0))],
        out_specs=pl.BlockSpec((ROWB, f), lambda i: (i, 0)),
        out_shape=jax.ShapeDtypeStruct((n, f), jnp.float32),
    )(xin, dinv)


def _tc_assemble(p0, p1, xprev, hprev, dinv, dinv2, feature_split, ck, hscale):
    n, f = xprev.shape
    nb = n // ROWB

    def body(p0_ref, p1_ref, xp_ref, hp_ref, d_ref, d2_ref,
             xk_ref, h_ref, t_ref):
        if feature_split:
            agg = jnp.concatenate([p0_ref[...], p1_ref[...]], axis=1)
        else:
            agg = p0_ref[...] + p1_ref[...]
        xk = d_ref[...] * agg + d2_ref[...] * xp_ref[...]
        xk_ref[...] = xk
        h_ref[...] = hscale * hp_ref[...] + ck * xk
        t_ref[...] = d_ref[...] * xk

    return pl.pallas_call(
        body,
        grid=(nb,),
        in_specs=[pl.BlockSpec((ROWB, 16), lambda i: (i, 0)),
                  pl.BlockSpec((ROWB, 16), lambda i: (i, 0)),
                  pl.BlockSpec((ROWB, f), lambda i: (i, 0)),
                  pl.BlockSpec((ROWB, f), lambda i: (i, 0)),
                  pl.BlockSpec((ROWB, 1), lambda i: (i, 0)),
                  pl.BlockSpec((ROWB, 1), lambda i: (i, 0))],
        out_specs=[pl.BlockSpec((ROWB, f), lambda i: (i, 0)),
                   pl.BlockSpec((ROWB, f), lambda i: (i, 0)),
                   pl.BlockSpec((ROWB, f), lambda i: (i, 0))],
        out_shape=[jax.ShapeDtypeStruct((n, f), jnp.float32),
                   jax.ShapeDtypeStruct((n, f), jnp.float32),
                   jax.ShapeDtypeStruct((n, f), jnp.float32)],
    )(p0, p1, xprev, hprev, dinv, dinv2)


def _rep8(v):
    return jnp.tile(v.reshape(1, -1), (8, 1))


def _lin_block(params, name, xin):
    w = params[name + "_W"]
    fi = w.shape[0]
    if xin.shape[1] != fi:
        pad = fi - xin.shape[1]
        xin = jnp.pad(xin, ((0, 0), (0, pad)))
    y, s1, s2 = _tc_linstats(xin, w, _rep8(params[name + "_b"]))
    n = xin.shape[0]
    mean = s1[0] / n
    var = s2[0] / n - mean * mean
    scale = params[name + "_bn_g"] * lax.rsqrt(var + 1e-5)
    shift = params[name + "_bn_b"] - mean * scale
    return _tc_bnact(y, _rep8(scale), _rep8(shift), _rep8(params[name + "_pr_a"]))


def _to_table(tab, feature_split):
    if feature_split:
        return jnp.concatenate([tab[:, :16], tab[:, 16:]], axis=0)
    return tab


def _ssg_block(params, name, xin, k_hops, srcs2, dst, zeros16, dinv, dinv2):
    n, f = xin.shape
    fsplit = f == 32
    ck = (1.0 - ALPHA) / k_hops
    tab = _tc_table(xin, dinv)
    table = _to_table(tab, fsplit)
    xk = xin
    h = xin
    for k in range(k_hops):
        p = _sc_prop(table, srcs2, dst, zeros16, n, fsplit)
        hscale = ALPHA if k == 0 else 1.0
        xk, h, tab = _tc_assemble(p[0], p[1], xk, h, dinv, dinv2,
                                  fsplit, ck, hscale)
        if k + 1 < k_hops:
            table = _to_table(tab, fsplit)
    w = params[name + "_W"]
    y, s1, s2 = _tc_linstats(h, w, _rep8(params[name + "_b"]))
    mean = s1[0] / n
    var = s2[0] / n - mean * mean
    scale = params[name + "_bn_g"] * lax.rsqrt(var + 1e-5)
    shift = params[name + "_bn_b"] - mean * scale
    return _tc_bnact(y, _rep8(scale), _rep8(shift), _rep8(params[name + "_pr_a"]))


def kernel(x, adj_matrix, params):
    n = x.shape[0]
    src = adj_matrix[0]
    dst = adj_matrix[1]
    srcs2 = jnp.concatenate([src, src + n])
    zeros16 = jnp.zeros((n, 16), jnp.float32)
    zeros_n = jnp.zeros((n,), jnp.float32)
    ones_c = jnp.ones((CHUNK,), jnp.float32)

    degp = _sc_degree(dst, zeros_n, ones_c)
    dinv, dinv2 = _tc_dinv(degp, n)

    x1 = _lin_block(params, "lin0", x)
    x2 = _lin_block(params, "lin1", x1)
    x3 = _ssg_block(params, "ssg2", x2, 3, srcs2, dst, zeros16, dinv, dinv2)
    x4 = _ssg_block(params, "ssg3", jnp.concatenate([x2, x3], axis=1), 4,
                    srcs2, dst, zeros16, dinv, dinv2)
    x5 = _ssg_block(params, "ssg4", jnp.concatenate([x3, x4], axis=1), 4,
                    srcs2, dst, zeros16, dinv, dinv2)
    x6 = _ssg_block(params, "ssg5", jnp.concatenate([x4, x5], axis=1), 3,
                    srcs2, dst, zeros16, dinv, dinv2)
    x7 = _lin_block(params, "lin6", jnp.concatenate([x2, x5, x6], axis=1))
    x8 = _lin_block(params, "lin7", jnp.concatenate([x1, x7], axis=1))
    return x8
